# Initial kernel scaffold; baseline (speedup 1.0000x reference)
import jax
import jax.numpy as jnp
from jax.experimental import pallas as pl


def kernel(node, edge_index, edge_attr, batch_ptr, eps, W1, b1, g1, be1, W2, b2, g2, be2, W3, b3, gn_weight, gn_bias, gn_mean_scale):
    raise NotImplementedError("write your pallas kernel here")


# speedup vs baseline: 5.0040x; 5.0040x over previous
import functools

import jax
import jax.numpy as jnp
from jax import lax
from jax.experimental import pallas as pl
from jax.experimental.pallas import tpu as pltpu
from jax.experimental.pallas import tpu_sc as plsc

N = 10000
D = 128
E = 320000
GPAD = 128

NC = 2
NS = 16
NW = NC * NS
CHUNK = 128
NCHUNK = 79
EPT = CHUNK * NCHUNK
EPAD = EPT * NW
RP = 10240
RT = RP // NS
RCH = RT // CHUNK

BLK = 1000
GRID = N // BLK


def _sc_agg_body(node_hbm, src_hbm, dst_hbm, out_hbm,
                 src_v, dst_v, rows_v, acc_sh, sem):
    c = lax.axis_index("c")
    s = lax.axis_index("s")
    wid = s * NC + c

    def _zrow(r, carry):
        for k in range(8):
            rows_v[r, pl.ds(k * 16, 16)] = jnp.zeros((16,), jnp.float32)
        return carry
    lax.fori_loop(0, CHUNK, _zrow, 0)
    for k in range(RCH):
        pltpu.sync_copy(rows_v, acc_sh.at[pl.ds(s * RT + k * CHUNK, CHUNK)])
    plsc.subcore_barrier()

    pltpu.sync_copy(src_hbm.at[wid], src_v)
    pltpu.sync_copy(dst_hbm.at[wid], dst_v)

    def _chunk(j, carry):
        pltpu.async_copy(node_hbm.at[src_v.at[j]], rows_v, sem).wait()
        pltpu.sync_copy(rows_v, acc_sh.at[dst_v.at[j]], add=True)
        return carry
    lax.fori_loop(0, NCHUNK, _chunk, 0)
    plsc.subcore_barrier()

    for k in range(RCH):
        pltpu.sync_copy(acc_sh.at[pl.ds(s * RT + k * CHUNK, CHUNK)], rows_v)
        pltpu.sync_copy(rows_v, out_hbm.at[c, pl.ds(s * RT + k * CHUNK, CHUNK)])


def _sc_aggregate(node, src3, dst3):
    mesh = plsc.VectorSubcoreMesh(core_axis_name="c", subcore_axis_name="s",
                                  num_cores=NC)
    f = pl.kernel(
        _sc_agg_body,
        mesh=mesh,
        out_type=jax.ShapeDtypeStruct((NC, RP, D), jnp.float32),
        scratch_types=[
            pltpu.VMEM((NCHUNK, CHUNK), jnp.int32),
            pltpu.VMEM((NCHUNK, CHUNK), jnp.int32),
            pltpu.VMEM((CHUNK, D), jnp.float32),
            pltpu.VMEM_SHARED((RP, D), jnp.float32),
            pltpu.SemaphoreType.DMA,
        ],
    )
    return f(node, src3, dst3)


def _ln(h, g, b):
    mu = jnp.mean(h, axis=-1, keepdims=True)
    d = h - mu
    var = jnp.mean(d * d, axis=-1, keepdims=True)
    return d * lax.rsqrt(var + 1e-5) * g + b


def _onehot(bp_ref):
    bp = bp_ref[0, 0]
    iot = lax.broadcasted_iota(jnp.int32, (BLK, GPAD), 1)
    return (bp[:, None] == iot).astype(jnp.float32)


def _mlp_stats_body(node_ref, a0_ref, a1_ref, bp_ref, epsv_ref,
                    W1_ref, b1_ref, g1_ref, be1_ref,
                    W2_ref, b2_ref, g2_ref, be2_ref,
                    W3_ref, b3_ref,
                    h3_ref, ssum_ref, ssq_ref, cmat_ref):
    i = pl.program_id(0)
    x = epsv_ref[...] * node_ref[...] + a0_ref[0] + a1_ref[0]
    h = jax.nn.relu(_ln(jnp.dot(x, W1_ref[...],
                                preferred_element_type=jnp.float32)
                        + b1_ref[...], g1_ref[...], be1_ref[...]))
    h = jax.nn.relu(_ln(jnp.dot(h, W2_ref[...],
                                preferred_element_type=jnp.float32)
                        + b2_ref[...], g2_ref[...], be2_ref[...]))
    h3 = jnp.dot(h, W3_ref[...], preferred_element_type=jnp.float32) \
        + b3_ref[...]
    h3_ref[...] = h3

    oh = _onehot(bp_ref)
    dn = (((0,), (0,)), ((), ()))

    @pl.when(i == 0)
    def _init():
        ssum_ref[...] = jnp.zeros_like(ssum_ref)
        ssq_ref[...] = jnp.zeros_like(ssq_ref)
        cmat_ref[...] = jnp.zeros_like(cmat_ref)

    ssum_ref[...] += lax.dot_general(oh, h3, dn,
                                     preferred_element_type=jnp.float32)
    ssq_ref[...] += lax.dot_general(oh, h3 * h3, dn,
                                    preferred_element_type=jnp.float32)
    cmat_ref[...] += lax.dot_general(oh, jnp.ones((BLK, D), jnp.float32), dn,
                                     preferred_element_type=jnp.float32)


def _gnorm_body(h3_ref, bp_ref, ssum_ref, ssq_ref, cmat_ref,
                gnw_ref, gnb_ref, gms_ref, out_ref):
    inv = 1.0 / jnp.maximum(cmat_ref[...], 1.0)
    mean = ssum_ref[...] * inv
    msq = ssq_ref[...] * inv
    s = gms_ref[...]
    A = s * mean
    var = msq - (2.0 * s - s * s) * (mean * mean)
    Binv = lax.rsqrt(var + 1e-5)
    oh = _onehot(bp_ref)
    rowA = jnp.dot(oh, A, preferred_element_type=jnp.float32)
    rowB = jnp.dot(oh, Binv, preferred_element_type=jnp.float32)
    out_ref[...] = jax.nn.relu(
        gnw_ref[...] * (h3_ref[...] - rowA) * rowB + gnb_ref[...])


def _row_spec():
    return pl.BlockSpec((BLK, D), lambda i: (i, 0))


def _full_spec(shape):
    nd = len(shape)
    return pl.BlockSpec(shape, lambda i: (0,) * nd)


def _bp_spec():
    return pl.BlockSpec((1, 1, BLK), lambda i: (i, 0, 0))


def kernel(node, edge_index, edge_attr, batch_ptr, eps,
           W1, b1, g1, be1, W2, b2, g2, be2, W3, b3,
           gn_weight, gn_bias, gn_mean_scale):
    del edge_attr
    src = edge_index[0].astype(jnp.int32)
    dst = edge_index[1].astype(jnp.int32)
    npad = EPAD - E
    src_p = jnp.concatenate([src, jnp.zeros((npad,), jnp.int32)])
    dst_p = jnp.concatenate([dst, jnp.full((npad,), N, jnp.int32)])
    src3 = src_p.reshape(NW, NCHUNK, CHUNK)
    dst3 = dst_p.reshape(NW, NCHUNK, CHUNK)

    agg2 = _sc_aggregate(node, src3, dst3)

    bp3 = batch_ptr.astype(jnp.int32).reshape(GRID, 1, BLK)
    epsv = jnp.full((1, D), 1.0 + eps, jnp.float32)
    r = lambda v: v.reshape(1, D)

    h3, ssum, ssq, cmat = pl.pallas_call(
        _mlp_stats_body,
        grid=(GRID,),
        in_specs=[
            _row_spec(),
            pl.BlockSpec((1, BLK, D), lambda i: (0, i, 0)),
            pl.BlockSpec((1, BLK, D), lambda i: (1, i, 0)),
            _bp_spec(),
            _full_spec((1, D)),
            _full_spec((D, D)), _full_spec((1, D)), _full_spec((1, D)),
            _full_spec((1, D)),
            _full_spec((D, D)), _full_spec((1, D)), _full_spec((1, D)),
            _full_spec((1, D)),
            _full_spec((D, D)), _full_spec((1, D)),
        ],
        out_specs=[
            _row_spec(),
            _full_spec((GPAD, D)),
            _full_spec((GPAD, D)),
            _full_spec((GPAD, D)),
        ],
        out_shape=[
            jax.ShapeDtypeStruct((N, D), jnp.float32),
            jax.ShapeDtypeStruct((GPAD, D), jnp.float32),
            jax.ShapeDtypeStruct((GPAD, D), jnp.float32),
            jax.ShapeDtypeStruct((GPAD, D), jnp.float32),
        ],
    )(node, agg2, agg2, bp3, epsv,
      W1, r(b1), r(g1), r(be1), W2, r(b2), r(g2), r(be2), W3, r(b3))

    out = pl.pallas_call(
        _gnorm_body,
        grid=(GRID,),
        in_specs=[
            _row_spec(),
            _bp_spec(),
            _full_spec((GPAD, D)),
            _full_spec((GPAD, D)),
            _full_spec((GPAD, D)),
            _full_spec((1, D)),
            _full_spec((1, D)),
            _full_spec((1, D)),
        ],
        out_specs=_row_spec(),
        out_shape=jax.ShapeDtypeStruct((N, D), jnp.float32),
    )(h3, bp3, ssum, ssq, cmat,
      r(gn_weight), r(gn_bias), r(gn_mean_scale))

    return out
